# baseline (speedup 1.0000x reference)
import functools

import jax
import jax.numpy as jnp
from jax import lax
from jax.experimental import pallas as pl
from jax.experimental.pallas import tpu as pltpu
from jax.experimental.pallas import tpu_sc as plsc

N = 50000
E = 800000
D_IN = 16
H = 128
L = 3

NC = 2
NS = 16
LANES = 16

NB = 8448
K = 6
NPAD = K * NB
G = 128
NCH = 2
EPAD = 819200
ROWS_ALL = EPAD // G
R_PER_TILE_SC = ROWS_ALL // NS

CNT = 51200
RB = 3168

NW = NC * NS
R_PER_TILE_ALL = ROWS_ALL // NW
CAPR = 201
STG = 272
DUMMY_OFF = NW * K * CAPR * G
TOTW = DUMMY_OFF + G

_mesh = plsc.VectorSubcoreMesh(
    core_axis_name="c", subcore_axis_name="s", num_cores=NC, num_subcores=NS)


def _zero_vmem_rows(ref, nrows):
    z = jnp.zeros((LANES,), jnp.float32)

    @pl.loop(0, nrows)
    def _(r):
        for j in range(H // LANES):
            ref[r, pl.ds(j * LANES, LANES)] = z


def _deg_body(dst_hbm, deg_hbm, dstv, ones_v, zeros_v, counts, sem):
    c = lax.axis_index("c")
    s = lax.axis_index("s")

    one = jnp.full((LANES,), 1.0, jnp.float32)
    zero = jnp.zeros((LANES,), jnp.float32)
    for j in range(G // LANES):
        ones_v[pl.ds(j * LANES, LANES)] = one

    @pl.loop(0, 3200 // LANES)
    def _(r):
        zeros_v[pl.ds(r * LANES, LANES)] = zero

    pltpu.sync_copy(zeros_v, counts.at[pl.ds(s * 3200, 3200)])
    plsc.subcore_barrier()

    @pl.loop(0, R_PER_TILE_SC // NCH)
    def _(j):
        row0 = s * R_PER_TILE_SC + j * NCH
        pltpu.sync_copy(dst_hbm.at[pl.ds(row0, NCH)], dstv)
        for g in range(NCH):
            pltpu.sync_copy(ones_v, counts.at[dstv.at[g]], add=True)

    plsc.subcore_barrier()
    off = c * (NPAD // 2) + s * (NPAD // 2 // NS)
    pltpu.sync_copy(counts.at[pl.ds(off, NPAD // 2 // NS)],
                    zeros_v.at[pl.ds(0, NPAD // 2 // NS)])
    pltpu.sync_copy(zeros_v.at[pl.ds(0, NPAD // 2 // NS)],
                    deg_hbm.at[pl.ds(off, NPAD // 2 // NS)])


_deg_kernel = functools.partial(
    pl.kernel,
    out_type=jax.ShapeDtypeStruct((NPAD,), jnp.float32),
    mesh=_mesh,
    scratch_types=[
        pltpu.VMEM((NCH, G), jnp.int32),
        pltpu.VMEM((G,), jnp.float32),
        pltpu.VMEM((3200,), jnp.float32),
        pltpu.VMEM_SHARED((CNT,), jnp.float32),
        pltpu.SemaphoreType.DMA,
    ],
)(_deg_body)


def _bin_body(src_hbm, dst_hbm, bsrc_hbm, bdst_hbm, cnts_hbm, srcv, dstv,
              stg_s, stg_d, cntv, sem):
    c = lax.axis_index("c")
    s = lax.axis_index("s")
    w = s * NC + c
    zero_s = jnp.zeros((), jnp.int32)
    init = ((zero_s,) * K, (zero_s,) * K)
    one16 = jnp.full((LANES,), 1, jnp.int32)
    zi = jnp.zeros((LANES,), jnp.int32)

    @pl.loop(0, R_PER_TILE_ALL, init_carry=init)
    def fin(r, carry):
        cnts = list(carry[0])
        wrs = list(carry[1])
        row = w * R_PER_TILE_ALL + r
        pltpu.sync_copy(src_hbm.at[row], srcv)
        pltpu.sync_copy(dst_hbm.at[row], dstv)
        for i in range(G // LANES):
            d = dstv[pl.ds(i * LANES, LANES)]
            sv = srcv[pl.ds(i * LANES, LANES)]
            for bb in range(K):
                lo = bb * NB
                m = (d >= lo) & (d < lo + NB)
                mi = jnp.where(m, one16, zi)
                pos = cnts[bb] + plsc.cumsum(mi) - 1
                plsc.store_scatter(stg_s, [bb * STG + pos], sv, mask=m)
                plsc.store_scatter(stg_d, [bb * STG + pos], d - lo, mask=m)
                cnts[bb] = pos[LANES - 1] + 1
        for bb in range(K):
            full = cnts[bb] >= G

            @pl.when(full)
            def _(bb=bb, wr=wrs[bb]):
                off = ((w * K + bb) * CAPR + wr) * G
                pltpu.sync_copy(stg_s.at[pl.ds(bb * STG, G)],
                                bsrc_hbm.at[pl.ds(off, G)])
                pltpu.sync_copy(stg_d.at[pl.ds(bb * STG, G)],
                                bdst_hbm.at[pl.ds(off, G)])
                for q in range(G // LANES):
                    stg_s[pl.ds(bb * STG + q * LANES, LANES)] = (
                        stg_s[pl.ds(bb * STG + G + q * LANES, LANES)])
                    stg_d[pl.ds(bb * STG + q * LANES, LANES)] = (
                        stg_d[pl.ds(bb * STG + G + q * LANES, LANES)])

            cnts[bb] = jnp.where(full, cnts[bb] - G, cnts[bb])
            wrs[bb] = jnp.where(full, wrs[bb] + 1, wrs[bb])
        return (tuple(cnts), tuple(wrs))

    iota = lax.broadcasted_iota(jnp.int32, (LANES,), 0)
    trash = jnp.full((LANES,), NB, jnp.int32)
    cntv[pl.ds(0, LANES)] = zi
    for bb in range(K):
        cnt = fin[0][bb]
        wr = fin[1][bb]
        for q in range(G // LANES):
            lp = jnp.full((LANES,), q * LANES, jnp.int32) + iota
            m = (lp >= cnt) & (lp < G)
            plsc.store_scatter(stg_s, [bb * STG + lp], zi, mask=m)
            plsc.store_scatter(stg_d, [bb * STG + lp], trash, mask=m)

        @pl.when(cnt > 0)
        def _(bb=bb, wr=wr):
            off = ((w * K + bb) * CAPR + wr) * G
            pltpu.sync_copy(stg_s.at[pl.ds(bb * STG, G)],
                            bsrc_hbm.at[pl.ds(off, G)])
            pltpu.sync_copy(stg_d.at[pl.ds(bb * STG, G)],
                            bdst_hbm.at[pl.ds(off, G)])

        nlists = jnp.where(cnt > 0, wr + 1, wr) + zi
        plsc.store_scatter(cntv, [iota], nlists, mask=(iota == bb))
    pltpu.sync_copy(cntv.at[pl.ds(0, 8)], cnts_hbm.at[pl.ds(w * 8, 8)])

    @pl.when(w == 0)
    def _():
        for q in range(G // LANES):
            stg_s[pl.ds(q * LANES, LANES)] = zi
            stg_d[pl.ds(q * LANES, LANES)] = trash
        pltpu.sync_copy(stg_s.at[pl.ds(0, G)],
                        bsrc_hbm.at[pl.ds(DUMMY_OFF, G)])
        pltpu.sync_copy(stg_d.at[pl.ds(0, G)],
                        bdst_hbm.at[pl.ds(DUMMY_OFF, G)])


_bin_kernel = functools.partial(
    pl.kernel,
    out_type=(jax.ShapeDtypeStruct((TOTW,), jnp.int32),
              jax.ShapeDtypeStruct((TOTW,), jnp.int32),
              jax.ShapeDtypeStruct((NW * 8,), jnp.int32)),
    mesh=_mesh,
    compiler_params=pltpu.CompilerParams(needs_layout_passes=False),
    scratch_types=[
        pltpu.VMEM((G,), jnp.int32),
        pltpu.VMEM((G,), jnp.int32),
        pltpu.VMEM((K * STG,), jnp.int32),
        pltpu.VMEM((K * STG,), jnp.int32),
        pltpu.VMEM((LANES,), jnp.int32),
        pltpu.SemaphoreType.DMA,
    ],
)(_bin_body)


def _spmm_body(bsrc_hbm, bdst_hbm, cnts_hbm, u_hbm, agg_hbm, srcv, dstv,
               rows, zrows, cntv, accum, gsem, ssem):
    c = lax.axis_index("c")
    s = lax.axis_index("s")

    _zero_vmem_rows(zrows, 64)
    pltpu.sync_copy(cnts_hbm, cntv)

    def zero_accum():
        for t in range(NB // NS // 64):
            pltpu.sync_copy(zrows,
                            accum.at[pl.ds(s * (NB // NS) + t * 64, 64)])
        pltpu.sync_copy(zrows.at[pl.ds(0, 16)],
                        accum.at[pl.ds(s * (NB // NS) + (NB // NS // 64) * 64, 16)])

        @pl.when(s == 0)
        def _():
            pltpu.sync_copy(zrows.at[pl.ds(0, 16)], accum.at[pl.ds(NB, 16)])

    for kb in range(K // NC):
        bb = c * (K // NC) + kb
        lo = bb * NB
        zero_accum()
        plsc.subcore_barrier()

        t0 = s * 2
        n0 = cntv[pl.ds(t0 * 8 + bb, 1)][0]
        n1 = cntv[pl.ds((t0 + 1) * 8 + bb, 1)][0]
        ntot = n0 + n1
        npair = (ntot + 2) // 3

        @pl.loop(0, npair)
        def _(jj):
            for p in range(3):
                jn = jj * 3 + p
                in0 = jn < n0
                tt = jnp.where(in0, t0, t0 + 1)
                jl = jnp.where(in0, jn, jn - n0)
                off = jnp.where(jn < ntot,
                                ((tt * K + bb) * CAPR + jl) * G, DUMMY_OFF)
                pltpu.sync_copy(bsrc_hbm.at[pl.ds(off, G)], srcv.at[p])
                pltpu.sync_copy(bdst_hbm.at[pl.ds(off, G)], dstv.at[p])
            gd = [pltpu.async_copy(u_hbm.at[srcv.at[p]],
                                   rows.at[pl.ds(p * G, G)], gsem)
                  for p in range(3)]
            sd = []
            for p in range(3):
                gd[p].wait()
                sd.append(pltpu.async_copy(rows.at[pl.ds(p * G, G)],
                                           accum.at[dstv.at[p]], ssem,
                                           add=True))
            for d_ in sd:
                d_.wait()

        plsc.subcore_barrier()
        for t in range(NB // NS // G):
            pltpu.sync_copy(accum.at[pl.ds(s * (NB // NS) + t * G, G)],
                            rows.at[pl.ds(0, G)])
            pltpu.sync_copy(rows.at[pl.ds(0, G)],
                            agg_hbm.at[pl.ds(lo + s * (NB // NS) + t * G, G)])
        tail = s * (NB // NS) + (NB // NS // G) * G
        pltpu.sync_copy(accum.at[pl.ds(tail, 16)], rows.at[pl.ds(0, 16)])
        pltpu.sync_copy(rows.at[pl.ds(0, 16)],
                        agg_hbm.at[pl.ds(lo + tail, 16)])
        if kb + 1 < K // NC:
            plsc.subcore_barrier()


_spmm_kernel = functools.partial(
    pl.kernel,
    out_type=jax.ShapeDtypeStruct((NPAD, H), jnp.float32),
    mesh=_mesh,
    scratch_types=[
        pltpu.VMEM((4, G), jnp.int32),
        pltpu.VMEM((4, G), jnp.int32),
        pltpu.VMEM((3 * G, H), jnp.float32),
        pltpu.VMEM((64, H), jnp.float32),
        pltpu.VMEM((NW * 8,), jnp.int32),
        pltpu.VMEM_SHARED((NB + 16, H), jnp.float32),
        pltpu.SemaphoreType.DMA,
        pltpu.SemaphoreType.DMA,
    ],
)(_spmm_body)


def _prep_body(x_ref, deg_ref, w_in_ref, b_in_ref, wc0_ref, u0_ref):
    dinv = lax.rsqrt(deg_ref[...] + 1.0)
    h0 = jnp.dot(x_ref[...], w_in_ref[...],
                 preferred_element_type=jnp.float32) + b_in_ref[...]
    u0_ref[...] = jnp.dot(h0, wc0_ref[...],
                          preferred_element_type=jnp.float32) * dinv


def _prep_call(x_pad, deg_col, W_in, b_in, Wc0):
    grid = (NPAD // RB,)
    return pl.pallas_call(
        _prep_body,
        grid=grid,
        in_specs=[
            pl.BlockSpec((RB, D_IN), lambda i: (i, 0)),
            pl.BlockSpec((RB, 1), lambda i: (i, 0)),
            pl.BlockSpec((D_IN, H), lambda i: (0, 0)),
            pl.BlockSpec((H,), lambda i: (0,)),
            pl.BlockSpec((H, H), lambda i: (0, 0)),
        ],
        out_specs=pl.BlockSpec((RB, H), lambda i: (i, 0)),
        out_shape=jax.ShapeDtypeStruct((NPAD, H), jnp.float32),
    )(x_pad, deg_col, W_in, b_in, Wc0)


def _post_body(has_res, has_next, agg_ref, u_ref, deg_ref, g_ref, bt_ref,
               bc_ref, *rest):
    idx = 0
    hprev_ref = rest[idx] if has_res else None
    idx += int(has_res)
    wcn_ref = rest[idx] if has_next else None
    idx += int(has_next)
    h_ref = rest[idx]
    un_ref = rest[idx + 1] if has_next else None

    dinv = lax.rsqrt(deg_ref[...] + 1.0)
    t = dinv * (agg_ref[...] + u_ref[...]) + bc_ref[...]
    mu = jnp.mean(t, axis=-1, keepdims=True)
    var = jnp.mean((t - mu) ** 2, axis=-1, keepdims=True)
    ln = (t - mu) * lax.rsqrt(var + 1e-5) * g_ref[...] + bt_ref[...]
    h = jnp.maximum(ln, 0.0)
    if has_res:
        h = h + hprev_ref[...]
    h_ref[...] = h
    if has_next:
        un_ref[...] = jnp.dot(h, wcn_ref[...],
                              preferred_element_type=jnp.float32) * dinv


def _post_call(agg, u, deg_col, gamma_i, beta_i, bc_i, h_prev=None,
               Wc_next=None):
    has_res = h_prev is not None
    has_next = Wc_next is not None
    grid = (NPAD // RB,)
    in_specs = [
        pl.BlockSpec((RB, H), lambda i: (i, 0)),
        pl.BlockSpec((RB, H), lambda i: (i, 0)),
        pl.BlockSpec((RB, 1), lambda i: (i, 0)),
        pl.BlockSpec((H,), lambda i: (0,)),
        pl.BlockSpec((H,), lambda i: (0,)),
        pl.BlockSpec((H,), lambda i: (0,)),
    ]
    args = [agg, u, deg_col, gamma_i, beta_i, bc_i]
    if has_res:
        in_specs.append(pl.BlockSpec((RB, H), lambda i: (i, 0)))
        args.append(h_prev)
    if has_next:
        in_specs.append(pl.BlockSpec((H, H), lambda i: (0, 0)))
        args.append(Wc_next)
    out_specs = [pl.BlockSpec((RB, H), lambda i: (i, 0))]
    out_shape = [jax.ShapeDtypeStruct((NPAD, H), jnp.float32)]
    if has_next:
        out_specs.append(pl.BlockSpec((RB, H), lambda i: (i, 0)))
        out_shape.append(jax.ShapeDtypeStruct((NPAD, H), jnp.float32))
    res = pl.pallas_call(
        functools.partial(_post_body, has_res, has_next),
        grid=grid,
        in_specs=in_specs,
        out_specs=out_specs,
        out_shape=out_shape,
    )(*args)
    return res if has_next else (res[0], None)


def kernel(x, edge_index, W_in, b_in, Wc, bc, gamma, beta):
    src = edge_index[0].astype(jnp.int32)
    dst = edge_index[1].astype(jnp.int32)
    src_p = jnp.concatenate(
        [src, jnp.zeros((EPAD - E,), jnp.int32)]).reshape(ROWS_ALL, G)
    dst_p = jnp.concatenate(
        [dst, jnp.full((EPAD - E,), NPAD, jnp.int32)]).reshape(ROWS_ALL, G)
    x_pad = jnp.pad(x, ((0, NPAD - N), (0, 0)))

    deg = _deg_kernel(dst_p)
    deg_col = deg.reshape(NPAD, 1)
    bsrc, bdst, cnts = _bin_kernel(src_p, dst_p)

    u = _prep_call(x_pad, deg_col, W_in, b_in, Wc[0])
    h = None
    for i in range(L):
        agg = _spmm_kernel(bsrc, bdst, cnts, u)
        h_prev = h if i > 0 else None
        Wc_next = Wc[i + 1] if i + 1 < L else None
        h, u = _post_call(agg, u, deg_col, gamma[i], beta[i], bc[i],
                          h_prev=h_prev, Wc_next=Wc_next)
    return h[:N]


# speedup vs baseline: 1.1432x; 1.1432x over previous
import functools

import jax
import jax.numpy as jnp
from jax import lax
from jax.experimental import pallas as pl
from jax.experimental.pallas import tpu as pltpu
from jax.experimental.pallas import tpu_sc as plsc

N = 50000
E = 800000
D_IN = 16
H = 128
L = 3

NC = 2
NS = 16
LANES = 16

NB = 8448
K = 6
NPAD = K * NB
G = 128
NCH = 2
EPAD = 819200
ROWS_ALL = EPAD // G
R_PER_TILE_SC = ROWS_ALL // NS

CNT = 51200
RB = 3168

NW = NC * NS
R_PER_TILE_ALL = ROWS_ALL // NW
CAPR = 201
STG = 272
DUMMY_OFF = NW * K * CAPR * G
TOTW = DUMMY_OFF + G

_mesh = plsc.VectorSubcoreMesh(
    core_axis_name="c", subcore_axis_name="s", num_cores=NC, num_subcores=NS)


def _zero_vmem_rows(ref, nrows):
    z = jnp.zeros((LANES,), jnp.float32)

    @pl.loop(0, nrows)
    def _(r):
        for j in range(H // LANES):
            ref[r, pl.ds(j * LANES, LANES)] = z


def _deg_body(dst_hbm, deg_hbm, dstv, ones_v, zeros_v, counts, sem):
    c = lax.axis_index("c")
    s = lax.axis_index("s")

    one = jnp.full((LANES,), 1.0, jnp.float32)
    zero = jnp.zeros((LANES,), jnp.float32)
    for j in range(G // LANES):
        ones_v[pl.ds(j * LANES, LANES)] = one

    @pl.loop(0, 3200 // LANES)
    def _(r):
        zeros_v[pl.ds(r * LANES, LANES)] = zero

    pltpu.sync_copy(zeros_v, counts.at[pl.ds(s * 3200, 3200)])
    plsc.subcore_barrier()

    @pl.loop(0, R_PER_TILE_SC // NCH)
    def _(j):
        row0 = s * R_PER_TILE_SC + j * NCH
        pltpu.sync_copy(dst_hbm.at[pl.ds(row0, NCH)], dstv)
        for g in range(NCH):
            pltpu.sync_copy(ones_v, counts.at[dstv.at[g]], add=True)

    plsc.subcore_barrier()
    off = c * (NPAD // 2) + s * (NPAD // 2 // NS)
    pltpu.sync_copy(counts.at[pl.ds(off, NPAD // 2 // NS)],
                    zeros_v.at[pl.ds(0, NPAD // 2 // NS)])
    pltpu.sync_copy(zeros_v.at[pl.ds(0, NPAD // 2 // NS)],
                    deg_hbm.at[pl.ds(off, NPAD // 2 // NS)])


_deg_kernel = functools.partial(
    pl.kernel,
    out_type=jax.ShapeDtypeStruct((NPAD,), jnp.float32),
    mesh=_mesh,
    scratch_types=[
        pltpu.VMEM((NCH, G), jnp.int32),
        pltpu.VMEM((G,), jnp.float32),
        pltpu.VMEM((3200,), jnp.float32),
        pltpu.VMEM_SHARED((CNT,), jnp.float32),
        pltpu.SemaphoreType.DMA,
    ],
)(_deg_body)


def _bin_body(src_hbm, dst_hbm, bsrc_hbm, bdst_hbm, cnts_hbm, srcv, dstv,
              stg_s, stg_d, cntv, sem):
    c = lax.axis_index("c")
    s = lax.axis_index("s")
    w = s * NC + c
    zero_s = jnp.zeros((), jnp.int32)
    init = ((zero_s,) * K, (zero_s,) * K)
    one16 = jnp.full((LANES,), 1, jnp.int32)
    zi = jnp.zeros((LANES,), jnp.int32)

    @pl.loop(0, R_PER_TILE_ALL, init_carry=init)
    def fin(r, carry):
        cnts = list(carry[0])
        wrs = list(carry[1])
        row = w * R_PER_TILE_ALL + r
        pltpu.sync_copy(src_hbm.at[row], srcv)
        pltpu.sync_copy(dst_hbm.at[row], dstv)
        for i in range(G // LANES):
            d = dstv[pl.ds(i * LANES, LANES)]
            sv = srcv[pl.ds(i * LANES, LANES)]
            for bb in range(K):
                lo = bb * NB
                m = (d >= lo) & (d < lo + NB)
                mi = jnp.where(m, one16, zi)
                pos = cnts[bb] + plsc.cumsum(mi) - 1
                plsc.store_scatter(stg_s, [bb * STG + pos], sv, mask=m)
                plsc.store_scatter(stg_d, [bb * STG + pos], d - lo, mask=m)
                cnts[bb] = pos[LANES - 1] + 1
        for bb in range(K):
            full = cnts[bb] >= G

            @pl.when(full)
            def _(bb=bb, wr=wrs[bb]):
                off = ((w * K + bb) * CAPR + wr) * G
                pltpu.sync_copy(stg_s.at[pl.ds(bb * STG, G)],
                                bsrc_hbm.at[pl.ds(off, G)])
                pltpu.sync_copy(stg_d.at[pl.ds(bb * STG, G)],
                                bdst_hbm.at[pl.ds(off, G)])
                for q in range(G // LANES):
                    stg_s[pl.ds(bb * STG + q * LANES, LANES)] = (
                        stg_s[pl.ds(bb * STG + G + q * LANES, LANES)])
                    stg_d[pl.ds(bb * STG + q * LANES, LANES)] = (
                        stg_d[pl.ds(bb * STG + G + q * LANES, LANES)])

            cnts[bb] = jnp.where(full, cnts[bb] - G, cnts[bb])
            wrs[bb] = jnp.where(full, wrs[bb] + 1, wrs[bb])
        return (tuple(cnts), tuple(wrs))

    iota = lax.broadcasted_iota(jnp.int32, (LANES,), 0)
    trash = jnp.full((LANES,), NB, jnp.int32)
    cntv[pl.ds(0, LANES)] = zi
    for bb in range(K):
        cnt = fin[0][bb]
        wr = fin[1][bb]
        for q in range(G // LANES):
            lp = jnp.full((LANES,), q * LANES, jnp.int32) + iota
            m = (lp >= cnt) & (lp < G)
            plsc.store_scatter(stg_s, [bb * STG + lp], zi, mask=m)
            plsc.store_scatter(stg_d, [bb * STG + lp], trash, mask=m)

        @pl.when(cnt > 0)
        def _(bb=bb, wr=wr):
            off = ((w * K + bb) * CAPR + wr) * G
            pltpu.sync_copy(stg_s.at[pl.ds(bb * STG, G)],
                            bsrc_hbm.at[pl.ds(off, G)])
            pltpu.sync_copy(stg_d.at[pl.ds(bb * STG, G)],
                            bdst_hbm.at[pl.ds(off, G)])

        nlists = jnp.where(cnt > 0, wr + 1, wr) + zi
        plsc.store_scatter(cntv, [iota], nlists, mask=(iota == bb))
    pltpu.sync_copy(cntv.at[pl.ds(0, 8)], cnts_hbm.at[pl.ds(w * 8, 8)])

    @pl.when(w == 0)
    def _():
        for q in range(G // LANES):
            stg_s[pl.ds(q * LANES, LANES)] = zi
            stg_d[pl.ds(q * LANES, LANES)] = trash
        pltpu.sync_copy(stg_s.at[pl.ds(0, G)],
                        bsrc_hbm.at[pl.ds(DUMMY_OFF, G)])
        pltpu.sync_copy(stg_d.at[pl.ds(0, G)],
                        bdst_hbm.at[pl.ds(DUMMY_OFF, G)])


_bin_kernel = functools.partial(
    pl.kernel,
    out_type=(jax.ShapeDtypeStruct((TOTW,), jnp.int32),
              jax.ShapeDtypeStruct((TOTW,), jnp.int32),
              jax.ShapeDtypeStruct((NW * 8,), jnp.int32)),
    mesh=_mesh,
    compiler_params=pltpu.CompilerParams(needs_layout_passes=False),
    scratch_types=[
        pltpu.VMEM((G,), jnp.int32),
        pltpu.VMEM((G,), jnp.int32),
        pltpu.VMEM((K * STG,), jnp.int32),
        pltpu.VMEM((K * STG,), jnp.int32),
        pltpu.VMEM((LANES,), jnp.int32),
        pltpu.SemaphoreType.DMA,
    ],
)(_bin_body)


def _spmm_body(bsrc_hbm, bdst_hbm, cnts_hbm, u_hbm, agg_hbm, srcv, dstv,
               rows, zrows, cntv, accum, gsem, ssem):
    c = lax.axis_index("c")
    s = lax.axis_index("s")

    _zero_vmem_rows(zrows, 64)
    pltpu.sync_copy(cnts_hbm, cntv)

    def zero_accum():
        for t in range(NB // NS // 64):
            pltpu.sync_copy(zrows,
                            accum.at[pl.ds(s * (NB // NS) + t * 64, 64)])
        pltpu.sync_copy(zrows.at[pl.ds(0, 16)],
                        accum.at[pl.ds(s * (NB // NS) + (NB // NS // 64) * 64, 16)])

        @pl.when(s == 0)
        def _():
            pltpu.sync_copy(zrows.at[pl.ds(0, 16)], accum.at[pl.ds(NB, 16)])

    for kb in range(K // NC):
        bb = c * (K // NC) + kb
        lo = bb * NB
        zero_accum()
        plsc.subcore_barrier()

        t0 = s * 2
        n0 = cntv[pl.ds(t0 * 8 + bb, 1)][0]
        n1 = cntv[pl.ds((t0 + 1) * 8 + bb, 1)][0]
        ntot = n0 + n1
        npair = (ntot + 1) // 2

        def idx_off(jn):
            in0 = jn < n0
            tt = jnp.where(in0, t0, t0 + 1)
            jl = jnp.where(in0, jn, jn - n0)
            return jnp.where(jn < ntot, ((tt * K + bb) * CAPR + jl) * G,
                             DUMMY_OFF)

        for p in range(2):
            off = idx_off(p)
            pltpu.sync_copy(bsrc_hbm.at[pl.ds(off, G)], srcv.at[p])
            pltpu.sync_copy(bdst_hbm.at[pl.ds(off, G)], dstv.at[p])

        @pl.loop(0, npair)
        def _(jj):
            gd = [pltpu.async_copy(u_hbm.at[srcv.at[p]],
                                   rows.at[pl.ds(p * G, G)], gsem)
                  for p in range(2)]
            sd = []
            for p in range(2):
                gd[p].wait()
                sd.append(pltpu.async_copy(rows.at[pl.ds(p * G, G)],
                                           accum.at[dstv.at[p]], ssem,
                                           add=True))
            noff = [idx_off((jj + 1) * 2 + p) for p in range(2)]
            for p in range(2):
                pltpu.sync_copy(bsrc_hbm.at[pl.ds(noff[p], G)], srcv.at[p])
            for p in range(2):
                sd[p].wait()
                pltpu.sync_copy(bdst_hbm.at[pl.ds(noff[p], G)], dstv.at[p])

        plsc.subcore_barrier()
        for t in range(NB // NS // G):
            pltpu.sync_copy(accum.at[pl.ds(s * (NB // NS) + t * G, G)],
                            rows.at[pl.ds(0, G)])
            pltpu.sync_copy(rows.at[pl.ds(0, G)],
                            agg_hbm.at[pl.ds(lo + s * (NB // NS) + t * G, G)])
        tail = s * (NB // NS) + (NB // NS // G) * G
        pltpu.sync_copy(accum.at[pl.ds(tail, 16)], rows.at[pl.ds(0, 16)])
        pltpu.sync_copy(rows.at[pl.ds(0, 16)],
                        agg_hbm.at[pl.ds(lo + tail, 16)])
        if kb + 1 < K // NC:
            plsc.subcore_barrier()


_spmm_kernel = functools.partial(
    pl.kernel,
    out_type=jax.ShapeDtypeStruct((NPAD, H), jnp.float32),
    mesh=_mesh,
    scratch_types=[
        pltpu.VMEM((2, G), jnp.int32),
        pltpu.VMEM((2, G), jnp.int32),
        pltpu.VMEM((2 * G, H), jnp.float32),
        pltpu.VMEM((64, H), jnp.float32),
        pltpu.VMEM((NW * 8,), jnp.int32),
        pltpu.VMEM_SHARED((NB + 16, H), jnp.float32),
        pltpu.SemaphoreType.DMA,
        pltpu.SemaphoreType.DMA,
    ],
)(_spmm_body)


def _prep_body(x_ref, deg_ref, w_in_ref, b_in_ref, wc0_ref, u0_ref):
    dinv = lax.rsqrt(deg_ref[...] + 1.0)
    h0 = jnp.dot(x_ref[...], w_in_ref[...],
                 preferred_element_type=jnp.float32) + b_in_ref[...]
    u0_ref[...] = jnp.dot(h0, wc0_ref[...],
                          preferred_element_type=jnp.float32) * dinv


def _prep_call(x_pad, deg_col, W_in, b_in, Wc0):
    grid = (NPAD // RB,)
    return pl.pallas_call(
        _prep_body,
        grid=grid,
        in_specs=[
            pl.BlockSpec((RB, D_IN), lambda i: (i, 0)),
            pl.BlockSpec((RB, 1), lambda i: (i, 0)),
            pl.BlockSpec((D_IN, H), lambda i: (0, 0)),
            pl.BlockSpec((H,), lambda i: (0,)),
            pl.BlockSpec((H, H), lambda i: (0, 0)),
        ],
        out_specs=pl.BlockSpec((RB, H), lambda i: (i, 0)),
        out_shape=jax.ShapeDtypeStruct((NPAD, H), jnp.float32),
    )(x_pad, deg_col, W_in, b_in, Wc0)


def _post_body(has_res, has_next, agg_ref, u_ref, deg_ref, g_ref, bt_ref,
               bc_ref, *rest):
    idx = 0
    hprev_ref = rest[idx] if has_res else None
    idx += int(has_res)
    wcn_ref = rest[idx] if has_next else None
    idx += int(has_next)
    h_ref = rest[idx]
    un_ref = rest[idx + 1] if has_next else None

    dinv = lax.rsqrt(deg_ref[...] + 1.0)
    t = dinv * (agg_ref[...] + u_ref[...]) + bc_ref[...]
    mu = jnp.mean(t, axis=-1, keepdims=True)
    var = jnp.mean((t - mu) ** 2, axis=-1, keepdims=True)
    ln = (t - mu) * lax.rsqrt(var + 1e-5) * g_ref[...] + bt_ref[...]
    h = jnp.maximum(ln, 0.0)
    if has_res:
        h = h + hprev_ref[...]
    h_ref[...] = h
    if has_next:
        un_ref[...] = jnp.dot(h, wcn_ref[...],
                              preferred_element_type=jnp.float32) * dinv


def _post_call(agg, u, deg_col, gamma_i, beta_i, bc_i, h_prev=None,
               Wc_next=None):
    has_res = h_prev is not None
    has_next = Wc_next is not None
    grid = (NPAD // RB,)
    in_specs = [
        pl.BlockSpec((RB, H), lambda i: (i, 0)),
        pl.BlockSpec((RB, H), lambda i: (i, 0)),
        pl.BlockSpec((RB, 1), lambda i: (i, 0)),
        pl.BlockSpec((H,), lambda i: (0,)),
        pl.BlockSpec((H,), lambda i: (0,)),
        pl.BlockSpec((H,), lambda i: (0,)),
    ]
    args = [agg, u, deg_col, gamma_i, beta_i, bc_i]
    if has_res:
        in_specs.append(pl.BlockSpec((RB, H), lambda i: (i, 0)))
        args.append(h_prev)
    if has_next:
        in_specs.append(pl.BlockSpec((H, H), lambda i: (0, 0)))
        args.append(Wc_next)
    out_specs = [pl.BlockSpec((RB, H), lambda i: (i, 0))]
    out_shape = [jax.ShapeDtypeStruct((NPAD, H), jnp.float32)]
    if has_next:
        out_specs.append(pl.BlockSpec((RB, H), lambda i: (i, 0)))
        out_shape.append(jax.ShapeDtypeStruct((NPAD, H), jnp.float32))
    res = pl.pallas_call(
        functools.partial(_post_body, has_res, has_next),
        grid=grid,
        in_specs=in_specs,
        out_specs=out_specs,
        out_shape=out_shape,
    )(*args)
    return res if has_next else (res[0], None)


def kernel(x, edge_index, W_in, b_in, Wc, bc, gamma, beta):
    src = edge_index[0].astype(jnp.int32)
    dst = edge_index[1].astype(jnp.int32)
    src_p = jnp.concatenate(
        [src, jnp.zeros((EPAD - E,), jnp.int32)]).reshape(ROWS_ALL, G)
    dst_p = jnp.concatenate(
        [dst, jnp.full((EPAD - E,), NPAD, jnp.int32)]).reshape(ROWS_ALL, G)
    x_pad = jnp.pad(x, ((0, NPAD - N), (0, 0)))

    deg = _deg_kernel(dst_p)
    deg_col = deg.reshape(NPAD, 1)
    bsrc, bdst, cnts = _bin_kernel(src_p, dst_p)

    u = _prep_call(x_pad, deg_col, W_in, b_in, Wc[0])
    h = None
    for i in range(L):
        agg = _spmm_kernel(bsrc, bdst, cnts, u)
        h_prev = h if i > 0 else None
        Wc_next = Wc[i + 1] if i + 1 < L else None
        h, u = _post_call(agg, u, deg_col, gamma[i], beta[i], bc[i],
                          h_prev=h_prev, Wc_next=Wc_next)
    return h[:N]


# speedup vs baseline: 1.1925x; 1.0432x over previous
import functools

import jax
import jax.numpy as jnp
from jax import lax
from jax.experimental import pallas as pl
from jax.experimental.pallas import tpu as pltpu
from jax.experimental.pallas import tpu_sc as plsc

N = 50000
E = 800000
D_IN = 16
H = 128
L = 3

NC = 2
NS = 16
LANES = 16

NB = 8448
K = 6
NPAD = K * NB
G = 128
NCH = 2
EPAD = 819200
ROWS_ALL = EPAD // G
R_PER_TILE_SC = ROWS_ALL // NS

CNT = 51200
RB = 3168

NW = NC * NS
R_PER_TILE_ALL = ROWS_ALL // NW
CAPR = 201
STG = 272
DUMMY_OFF = NW * K * CAPR * G
TOTW = DUMMY_OFF + G

_mesh = plsc.VectorSubcoreMesh(
    core_axis_name="c", subcore_axis_name="s", num_cores=NC, num_subcores=NS)


def _zero_vmem_rows(ref, nrows):
    z = jnp.zeros((LANES,), jnp.float32)

    @pl.loop(0, nrows)
    def _(r):
        for j in range(H // LANES):
            ref[r, pl.ds(j * LANES, LANES)] = z


def _deg_body(dst_hbm, deg_hbm, dstv, ones_v, zeros_v, counts, sem):
    c = lax.axis_index("c")
    s = lax.axis_index("s")

    one = jnp.full((LANES,), 1.0, jnp.float32)
    zero = jnp.zeros((LANES,), jnp.float32)
    for j in range(G // LANES):
        ones_v[pl.ds(j * LANES, LANES)] = one

    @pl.loop(0, 3200 // LANES)
    def _(r):
        zeros_v[pl.ds(r * LANES, LANES)] = zero

    pltpu.sync_copy(zeros_v, counts.at[pl.ds(s * 3200, 3200)])
    plsc.subcore_barrier()

    @pl.loop(0, R_PER_TILE_SC // NCH)
    def _(j):
        row0 = s * R_PER_TILE_SC + j * NCH
        pltpu.sync_copy(dst_hbm.at[pl.ds(row0, NCH)], dstv)
        for g in range(NCH):
            pltpu.sync_copy(ones_v, counts.at[dstv.at[g]], add=True)

    plsc.subcore_barrier()
    off = c * (NPAD // 2) + s * (NPAD // 2 // NS)
    pltpu.sync_copy(counts.at[pl.ds(off, NPAD // 2 // NS)],
                    zeros_v.at[pl.ds(0, NPAD // 2 // NS)])
    pltpu.sync_copy(zeros_v.at[pl.ds(0, NPAD // 2 // NS)],
                    deg_hbm.at[pl.ds(off, NPAD // 2 // NS)])


_deg_kernel = functools.partial(
    pl.kernel,
    out_type=jax.ShapeDtypeStruct((NPAD,), jnp.float32),
    mesh=_mesh,
    scratch_types=[
        pltpu.VMEM((NCH, G), jnp.int32),
        pltpu.VMEM((G,), jnp.float32),
        pltpu.VMEM((3200,), jnp.float32),
        pltpu.VMEM_SHARED((CNT,), jnp.float32),
        pltpu.SemaphoreType.DMA,
    ],
)(_deg_body)


def _bin_body(src_hbm, dst_hbm, bsrc_hbm, bdst_hbm, cnts_hbm, srcv, dstv,
              stg_s, stg_d, cntv, sem):
    c = lax.axis_index("c")
    s = lax.axis_index("s")
    w = s * NC + c
    zero_s = jnp.zeros((), jnp.int32)
    init = ((zero_s,) * K, (zero_s,) * K)
    one16 = jnp.full((LANES,), 1, jnp.int32)
    zi = jnp.zeros((LANES,), jnp.int32)

    base = w * R_PER_TILE_ALL
    pltpu.async_copy(src_hbm.at[base], srcv.at[0], sem)
    pltpu.async_copy(dst_hbm.at[base], dstv.at[0], sem)

    @pl.loop(0, R_PER_TILE_ALL // 2, init_carry=init)
    def fin(r2, carry):
        cnts = list(carry[0])
        wrs = list(carry[1])
        row_a = base + 2 * r2
        row_b = row_a + 1
        row_n = jnp.where(row_a + 2 >= ROWS_ALL, 0, row_a + 2)

        def compact(buf, cnts):
            for i in range(G // LANES):
                d = dstv[buf, pl.ds(i * LANES, LANES)]
                sv = srcv[buf, pl.ds(i * LANES, LANES)]
                for bb in range(K):
                    lo = bb * NB
                    m = (d >= lo) & (d < lo + NB)
                    mi = jnp.where(m, one16, zi)
                    pos = cnts[bb] + plsc.cumsum(mi) - 1
                    plsc.store_scatter(stg_s, [bb * STG + pos], sv, mask=m)
                    plsc.store_scatter(stg_d, [bb * STG + pos], d - lo,
                                       mask=m)
                    cnts[bb] = pos[LANES - 1] + 1
            return cnts

        pltpu.make_async_copy(src_hbm.at[row_a], srcv.at[0], sem).wait()
        pltpu.make_async_copy(dst_hbm.at[row_a], dstv.at[0], sem).wait()
        gb0 = pltpu.async_copy(src_hbm.at[row_b], srcv.at[1], sem)
        gb1 = pltpu.async_copy(dst_hbm.at[row_b], dstv.at[1], sem)
        cnts = compact(0, cnts)
        gb0.wait()
        gb1.wait()
        pltpu.async_copy(src_hbm.at[row_n], srcv.at[0], sem)
        pltpu.async_copy(dst_hbm.at[row_n], dstv.at[0], sem)
        cnts = compact(1, cnts)
        for bb in range(K):
            full = cnts[bb] >= G

            @pl.when(full)
            def _(bb=bb, wr=wrs[bb]):
                off = ((w * K + bb) * CAPR + wr) * G
                pltpu.sync_copy(stg_s.at[pl.ds(bb * STG, G)],
                                bsrc_hbm.at[pl.ds(off, G)])
                pltpu.sync_copy(stg_d.at[pl.ds(bb * STG, G)],
                                bdst_hbm.at[pl.ds(off, G)])
                for q in range(G // LANES):
                    stg_s[pl.ds(bb * STG + q * LANES, LANES)] = (
                        stg_s[pl.ds(bb * STG + G + q * LANES, LANES)])
                    stg_d[pl.ds(bb * STG + q * LANES, LANES)] = (
                        stg_d[pl.ds(bb * STG + G + q * LANES, LANES)])

            cnts[bb] = jnp.where(full, cnts[bb] - G, cnts[bb])
            wrs[bb] = jnp.where(full, wrs[bb] + 1, wrs[bb])
        return (tuple(cnts), tuple(wrs))

    pltpu.make_async_copy(src_hbm.at[0], srcv.at[0], sem).wait()
    pltpu.make_async_copy(dst_hbm.at[0], dstv.at[0], sem).wait()

    iota = lax.broadcasted_iota(jnp.int32, (LANES,), 0)
    trash = jnp.full((LANES,), NB, jnp.int32)
    cntv[pl.ds(0, LANES)] = zi
    for bb in range(K):
        cnt = fin[0][bb]
        wr = fin[1][bb]
        for q in range(G // LANES):
            lp = jnp.full((LANES,), q * LANES, jnp.int32) + iota
            m = (lp >= cnt) & (lp < G)
            plsc.store_scatter(stg_s, [bb * STG + lp], zi, mask=m)
            plsc.store_scatter(stg_d, [bb * STG + lp], trash, mask=m)

        @pl.when(cnt > 0)
        def _(bb=bb, wr=wr):
            off = ((w * K + bb) * CAPR + wr) * G
            pltpu.sync_copy(stg_s.at[pl.ds(bb * STG, G)],
                            bsrc_hbm.at[pl.ds(off, G)])
            pltpu.sync_copy(stg_d.at[pl.ds(bb * STG, G)],
                            bdst_hbm.at[pl.ds(off, G)])

        nlists = jnp.where(cnt > 0, wr + 1, wr) + zi
        plsc.store_scatter(cntv, [iota], nlists, mask=(iota == bb))
    pltpu.sync_copy(cntv.at[pl.ds(0, 8)], cnts_hbm.at[pl.ds(w * 8, 8)])

    @pl.when(w == 0)
    def _():
        for q in range(G // LANES):
            stg_s[pl.ds(q * LANES, LANES)] = zi
            stg_d[pl.ds(q * LANES, LANES)] = trash
        pltpu.sync_copy(stg_s.at[pl.ds(0, G)],
                        bsrc_hbm.at[pl.ds(DUMMY_OFF, G)])
        pltpu.sync_copy(stg_d.at[pl.ds(0, G)],
                        bdst_hbm.at[pl.ds(DUMMY_OFF, G)])


_bin_kernel = functools.partial(
    pl.kernel,
    out_type=(jax.ShapeDtypeStruct((TOTW,), jnp.int32),
              jax.ShapeDtypeStruct((TOTW,), jnp.int32),
              jax.ShapeDtypeStruct((NW * 8,), jnp.int32)),
    mesh=_mesh,
    compiler_params=pltpu.CompilerParams(needs_layout_passes=False),
    scratch_types=[
        pltpu.VMEM((2, G), jnp.int32),
        pltpu.VMEM((2, G), jnp.int32),
        pltpu.VMEM((K * STG,), jnp.int32),
        pltpu.VMEM((K * STG,), jnp.int32),
        pltpu.VMEM((LANES,), jnp.int32),
        pltpu.SemaphoreType.DMA,
    ],
)(_bin_body)


def _spmm_body(bsrc_hbm, bdst_hbm, cnts_hbm, u_hbm, agg_hbm, srcv, dstv,
               rows, zrows, cntv, accum, gsem, ssem):
    c = lax.axis_index("c")
    s = lax.axis_index("s")

    _zero_vmem_rows(zrows, 64)
    pltpu.sync_copy(cnts_hbm, cntv)

    def zero_accum():
        for t in range(NB // NS // 64):
            pltpu.sync_copy(zrows,
                            accum.at[pl.ds(s * (NB // NS) + t * 64, 64)])
        pltpu.sync_copy(zrows.at[pl.ds(0, 16)],
                        accum.at[pl.ds(s * (NB // NS) + (NB // NS // 64) * 64, 16)])

        @pl.when(s == 0)
        def _():
            pltpu.sync_copy(zrows.at[pl.ds(0, 16)], accum.at[pl.ds(NB, 16)])

    for kb in range(K // NC):
        bb = c * (K // NC) + kb
        lo = bb * NB
        zero_accum()
        plsc.subcore_barrier()

        t0 = s * 2
        n0 = cntv[pl.ds(t0 * 8 + bb, 1)][0]
        n1 = cntv[pl.ds((t0 + 1) * 8 + bb, 1)][0]
        ntot = n0 + n1
        npair = (ntot + 1) // 2

        def idx_off(jn):
            in0 = jn < n0
            tt = jnp.where(in0, t0, t0 + 1)
            jl = jnp.where(in0, jn, jn - n0)
            return jnp.where(jn < ntot, ((tt * K + bb) * CAPR + jl) * G,
                             DUMMY_OFF)

        for p in range(2):
            off = idx_off(p)
            pltpu.sync_copy(bsrc_hbm.at[pl.ds(off, G)], srcv.at[p])
            pltpu.sync_copy(bdst_hbm.at[pl.ds(off, G)], dstv.at[p])

        @pl.loop(0, npair)
        def _(jj):
            gd = [pltpu.async_copy(u_hbm.at[srcv.at[p]],
                                   rows.at[pl.ds(p * G, G)], gsem)
                  for p in range(2)]
            sd = []
            for p in range(2):
                gd[p].wait()
                sd.append(pltpu.async_copy(rows.at[pl.ds(p * G, G)],
                                           accum.at[dstv.at[p]], ssem,
                                           add=True))
            noff = [idx_off((jj + 1) * 2 + p) for p in range(2)]
            for p in range(2):
                pltpu.sync_copy(bsrc_hbm.at[pl.ds(noff[p], G)], srcv.at[p])
            for p in range(2):
                sd[p].wait()
                pltpu.sync_copy(bdst_hbm.at[pl.ds(noff[p], G)], dstv.at[p])

        plsc.subcore_barrier()
        for t in range(NB // NS // G):
            pltpu.sync_copy(accum.at[pl.ds(s * (NB // NS) + t * G, G)],
                            rows.at[pl.ds(0, G)])
            pltpu.sync_copy(rows.at[pl.ds(0, G)],
                            agg_hbm.at[pl.ds(lo + s * (NB // NS) + t * G, G)])
        tail = s * (NB // NS) + (NB // NS // G) * G
        pltpu.sync_copy(accum.at[pl.ds(tail, 16)], rows.at[pl.ds(0, 16)])
        pltpu.sync_copy(rows.at[pl.ds(0, 16)],
                        agg_hbm.at[pl.ds(lo + tail, 16)])
        if kb + 1 < K // NC:
            plsc.subcore_barrier()


_spmm_kernel = functools.partial(
    pl.kernel,
    out_type=jax.ShapeDtypeStruct((NPAD, H), jnp.float32),
    mesh=_mesh,
    scratch_types=[
        pltpu.VMEM((2, G), jnp.int32),
        pltpu.VMEM((2, G), jnp.int32),
        pltpu.VMEM((2 * G, H), jnp.float32),
        pltpu.VMEM((64, H), jnp.float32),
        pltpu.VMEM((NW * 8,), jnp.int32),
        pltpu.VMEM_SHARED((NB + 16, H), jnp.float32),
        pltpu.SemaphoreType.DMA,
        pltpu.SemaphoreType.DMA,
    ],
)(_spmm_body)


def _prep_body(x_ref, deg_ref, w_in_ref, b_in_ref, wc0_ref, u0_ref):
    dinv = lax.rsqrt(deg_ref[...] + 1.0)
    h0 = jnp.dot(x_ref[...], w_in_ref[...],
                 preferred_element_type=jnp.float32) + b_in_ref[...]
    u0_ref[...] = jnp.dot(h0, wc0_ref[...],
                          preferred_element_type=jnp.float32) * dinv


def _prep_call(x_pad, deg_col, W_in, b_in, Wc0):
    grid = (NPAD // RB,)
    return pl.pallas_call(
        _prep_body,
        grid=grid,
        in_specs=[
            pl.BlockSpec((RB, D_IN), lambda i: (i, 0)),
            pl.BlockSpec((RB, 1), lambda i: (i, 0)),
            pl.BlockSpec((D_IN, H), lambda i: (0, 0)),
            pl.BlockSpec((H,), lambda i: (0,)),
            pl.BlockSpec((H, H), lambda i: (0, 0)),
        ],
        out_specs=pl.BlockSpec((RB, H), lambda i: (i, 0)),
        out_shape=jax.ShapeDtypeStruct((NPAD, H), jnp.float32),
    )(x_pad, deg_col, W_in, b_in, Wc0)


def _post_body(has_res, has_next, agg_ref, u_ref, deg_ref, g_ref, bt_ref,
               bc_ref, *rest):
    idx = 0
    hprev_ref = rest[idx] if has_res else None
    idx += int(has_res)
    wcn_ref = rest[idx] if has_next else None
    idx += int(has_next)
    h_ref = rest[idx]
    un_ref = rest[idx + 1] if has_next else None

    dinv = lax.rsqrt(deg_ref[...] + 1.0)
    t = dinv * (agg_ref[...] + u_ref[...]) + bc_ref[...]
    mu = jnp.mean(t, axis=-1, keepdims=True)
    var = jnp.mean((t - mu) ** 2, axis=-1, keepdims=True)
    ln = (t - mu) * lax.rsqrt(var + 1e-5) * g_ref[...] + bt_ref[...]
    h = jnp.maximum(ln, 0.0)
    if has_res:
        h = h + hprev_ref[...]
    h_ref[...] = h
    if has_next:
        un_ref[...] = jnp.dot(h, wcn_ref[...],
                              preferred_element_type=jnp.float32) * dinv


def _post_call(agg, u, deg_col, gamma_i, beta_i, bc_i, h_prev=None,
               Wc_next=None):
    has_res = h_prev is not None
    has_next = Wc_next is not None
    grid = (NPAD // RB,)
    in_specs = [
        pl.BlockSpec((RB, H), lambda i: (i, 0)),
        pl.BlockSpec((RB, H), lambda i: (i, 0)),
        pl.BlockSpec((RB, 1), lambda i: (i, 0)),
        pl.BlockSpec((H,), lambda i: (0,)),
        pl.BlockSpec((H,), lambda i: (0,)),
        pl.BlockSpec((H,), lambda i: (0,)),
    ]
    args = [agg, u, deg_col, gamma_i, beta_i, bc_i]
    if has_res:
        in_specs.append(pl.BlockSpec((RB, H), lambda i: (i, 0)))
        args.append(h_prev)
    if has_next:
        in_specs.append(pl.BlockSpec((H, H), lambda i: (0, 0)))
        args.append(Wc_next)
    out_specs = [pl.BlockSpec((RB, H), lambda i: (i, 0))]
    out_shape = [jax.ShapeDtypeStruct((NPAD, H), jnp.float32)]
    if has_next:
        out_specs.append(pl.BlockSpec((RB, H), lambda i: (i, 0)))
        out_shape.append(jax.ShapeDtypeStruct((NPAD, H), jnp.float32))
    res = pl.pallas_call(
        functools.partial(_post_body, has_res, has_next),
        grid=grid,
        in_specs=in_specs,
        out_specs=out_specs,
        out_shape=out_shape,
    )(*args)
    return res if has_next else (res[0], None)


def kernel(x, edge_index, W_in, b_in, Wc, bc, gamma, beta):
    src = edge_index[0].astype(jnp.int32)
    dst = edge_index[1].astype(jnp.int32)
    src_p = jnp.concatenate(
        [src, jnp.zeros((EPAD - E,), jnp.int32)]).reshape(ROWS_ALL, G)
    dst_p = jnp.concatenate(
        [dst, jnp.full((EPAD - E,), NPAD, jnp.int32)]).reshape(ROWS_ALL, G)
    x_pad = jnp.pad(x, ((0, NPAD - N), (0, 0)))

    deg = _deg_kernel(dst_p)
    deg_col = deg.reshape(NPAD, 1)
    bsrc, bdst, cnts = _bin_kernel(src_p, dst_p)

    u = _prep_call(x_pad, deg_col, W_in, b_in, Wc[0])
    h = None
    for i in range(L):
        agg = _spmm_kernel(bsrc, bdst, cnts, u)
        h_prev = h if i > 0 else None
        Wc_next = Wc[i + 1] if i + 1 < L else None
        h, u = _post_call(agg, u, deg_col, gamma[i], beta[i], bc[i],
                          h_prev=h_prev, Wc_next=Wc_next)
    return h[:N]


# speedup vs baseline: 1.2047x; 1.0102x over previous
import functools

import jax
import jax.numpy as jnp
from jax import lax
from jax.experimental import pallas as pl
from jax.experimental.pallas import tpu as pltpu
from jax.experimental.pallas import tpu_sc as plsc

N = 50000
E = 800000
D_IN = 16
H = 128
L = 3

NC = 2
NS = 16
LANES = 16

NB = 8448
K = 6
NPAD = K * NB
G = 128
NCH = 2
EPAD = 819200
ROWS_ALL = EPAD // G
R_PER_TILE_SC = ROWS_ALL // NS

CNT = 51200
RB = 3168

NW = NC * NS
R_PER_TILE_ALL = ROWS_ALL // NW
CAPR = 201
STG = 272
DUMMY_OFF = NW * K * CAPR * G
TOTW = DUMMY_OFF + G

_mesh = plsc.VectorSubcoreMesh(
    core_axis_name="c", subcore_axis_name="s", num_cores=NC, num_subcores=NS)


def _zero_vmem_rows(ref, nrows):
    z = jnp.zeros((LANES,), jnp.float32)

    @pl.loop(0, nrows)
    def _(r):
        for j in range(H // LANES):
            ref[r, pl.ds(j * LANES, LANES)] = z


def _deg_body(dst_hbm, deg_hbm, dstv, ones_v, zeros_v, counts, sem):
    c = lax.axis_index("c")
    s = lax.axis_index("s")

    one = jnp.full((LANES,), 1.0, jnp.float32)
    zero = jnp.zeros((LANES,), jnp.float32)
    for j in range(G // LANES):
        ones_v[pl.ds(j * LANES, LANES)] = one

    @pl.loop(0, 3200 // LANES)
    def _(r):
        zeros_v[pl.ds(r * LANES, LANES)] = zero

    pltpu.sync_copy(zeros_v, counts.at[pl.ds(s * 3200, 3200)])
    plsc.subcore_barrier()

    dbase = s * R_PER_TILE_SC
    pltpu.async_copy(dst_hbm.at[pl.ds(dbase, NCH)],
                     dstv.at[pl.ds(0, NCH)], sem)

    @pl.loop(0, R_PER_TILE_SC // NCH // 2)
    def _(j):
        row_a = dbase + j * 2 * NCH
        row_b = row_a + NCH
        row_n = jnp.where(row_a + 2 * NCH >= ROWS_ALL, 0, row_a + 2 * NCH)
        pltpu.make_async_copy(dst_hbm.at[pl.ds(row_a, NCH)],
                              dstv.at[pl.ds(0, NCH)], sem).wait()
        gb = pltpu.async_copy(dst_hbm.at[pl.ds(row_b, NCH)],
                              dstv.at[pl.ds(NCH, NCH)], sem)
        for g in range(NCH):
            pltpu.sync_copy(ones_v, counts.at[dstv.at[g]], add=True)
        gb.wait()
        pltpu.async_copy(dst_hbm.at[pl.ds(row_n, NCH)],
                         dstv.at[pl.ds(0, NCH)], sem)
        for g in range(NCH):
            pltpu.sync_copy(ones_v, counts.at[dstv.at[NCH + g]], add=True)

    pltpu.make_async_copy(dst_hbm.at[pl.ds(0, NCH)],
                          dstv.at[pl.ds(0, NCH)], sem).wait()
    plsc.subcore_barrier()
    off = c * (NPAD // 2) + s * (NPAD // 2 // NS)
    pltpu.sync_copy(counts.at[pl.ds(off, NPAD // 2 // NS)],
                    zeros_v.at[pl.ds(0, NPAD // 2 // NS)])
    pltpu.sync_copy(zeros_v.at[pl.ds(0, NPAD // 2 // NS)],
                    deg_hbm.at[pl.ds(off, NPAD // 2 // NS)])


_deg_kernel = functools.partial(
    pl.kernel,
    out_type=jax.ShapeDtypeStruct((NPAD,), jnp.float32),
    mesh=_mesh,
    scratch_types=[
        pltpu.VMEM((2 * NCH, G), jnp.int32),
        pltpu.VMEM((G,), jnp.float32),
        pltpu.VMEM((3200,), jnp.float32),
        pltpu.VMEM_SHARED((CNT,), jnp.float32),
        pltpu.SemaphoreType.DMA,
    ],
)(_deg_body)


def _bin_body(src_hbm, dst_hbm, bsrc_hbm, bdst_hbm, cnts_hbm, srcv, dstv,
              stg_s, stg_d, cntv, sem):
    c = lax.axis_index("c")
    s = lax.axis_index("s")
    w = s * NC + c
    zero_s = jnp.zeros((), jnp.int32)
    init = ((zero_s,) * K, (zero_s,) * K)
    one16 = jnp.full((LANES,), 1, jnp.int32)
    zi = jnp.zeros((LANES,), jnp.int32)

    base = w * R_PER_TILE_ALL
    pltpu.async_copy(src_hbm.at[base], srcv.at[0], sem)
    pltpu.async_copy(dst_hbm.at[base], dstv.at[0], sem)

    @pl.loop(0, R_PER_TILE_ALL // 2, init_carry=init)
    def fin(r2, carry):
        cnts = list(carry[0])
        wrs = list(carry[1])
        row_a = base + 2 * r2
        row_b = row_a + 1
        row_n = jnp.where(row_a + 2 >= ROWS_ALL, 0, row_a + 2)

        def compact(buf, cnts):
            for i in range(G // LANES):
                d = dstv[buf, pl.ds(i * LANES, LANES)]
                sv = srcv[buf, pl.ds(i * LANES, LANES)]
                for bb in range(K):
                    lo = bb * NB
                    m = (d >= lo) & (d < lo + NB)
                    mi = jnp.where(m, one16, zi)
                    pos = cnts[bb] + plsc.cumsum(mi) - 1
                    plsc.store_scatter(stg_s, [bb * STG + pos], sv, mask=m)
                    plsc.store_scatter(stg_d, [bb * STG + pos], d - lo,
                                       mask=m)
                    cnts[bb] = pos[LANES - 1] + 1
            return cnts

        pltpu.make_async_copy(src_hbm.at[row_a], srcv.at[0], sem).wait()
        pltpu.make_async_copy(dst_hbm.at[row_a], dstv.at[0], sem).wait()
        gb0 = pltpu.async_copy(src_hbm.at[row_b], srcv.at[1], sem)
        gb1 = pltpu.async_copy(dst_hbm.at[row_b], dstv.at[1], sem)
        cnts = compact(0, cnts)
        gb0.wait()
        gb1.wait()
        pltpu.async_copy(src_hbm.at[row_n], srcv.at[0], sem)
        pltpu.async_copy(dst_hbm.at[row_n], dstv.at[0], sem)
        cnts = compact(1, cnts)
        for bb in range(K):
            full = cnts[bb] >= G

            @pl.when(full)
            def _(bb=bb, wr=wrs[bb]):
                off = ((w * K + bb) * CAPR + wr) * G
                pltpu.sync_copy(stg_s.at[pl.ds(bb * STG, G)],
                                bsrc_hbm.at[pl.ds(off, G)])
                pltpu.sync_copy(stg_d.at[pl.ds(bb * STG, G)],
                                bdst_hbm.at[pl.ds(off, G)])
                for q in range(G // LANES):
                    stg_s[pl.ds(bb * STG + q * LANES, LANES)] = (
                        stg_s[pl.ds(bb * STG + G + q * LANES, LANES)])
                    stg_d[pl.ds(bb * STG + q * LANES, LANES)] = (
                        stg_d[pl.ds(bb * STG + G + q * LANES, LANES)])

            cnts[bb] = jnp.where(full, cnts[bb] - G, cnts[bb])
            wrs[bb] = jnp.where(full, wrs[bb] + 1, wrs[bb])
        return (tuple(cnts), tuple(wrs))

    pltpu.make_async_copy(src_hbm.at[0], srcv.at[0], sem).wait()
    pltpu.make_async_copy(dst_hbm.at[0], dstv.at[0], sem).wait()

    iota = lax.broadcasted_iota(jnp.int32, (LANES,), 0)
    trash = jnp.full((LANES,), NB, jnp.int32)
    cntv[pl.ds(0, LANES)] = zi
    for bb in range(K):
        cnt = fin[0][bb]
        wr = fin[1][bb]
        for q in range(G // LANES):
            lp = jnp.full((LANES,), q * LANES, jnp.int32) + iota
            m = (lp >= cnt) & (lp < G)
            plsc.store_scatter(stg_s, [bb * STG + lp], zi, mask=m)
            plsc.store_scatter(stg_d, [bb * STG + lp], trash, mask=m)

        @pl.when(cnt > 0)
        def _(bb=bb, wr=wr):
            off = ((w * K + bb) * CAPR + wr) * G
            pltpu.sync_copy(stg_s.at[pl.ds(bb * STG, G)],
                            bsrc_hbm.at[pl.ds(off, G)])
            pltpu.sync_copy(stg_d.at[pl.ds(bb * STG, G)],
                            bdst_hbm.at[pl.ds(off, G)])

        nlists = jnp.where(cnt > 0, wr + 1, wr) + zi
        plsc.store_scatter(cntv, [iota], nlists, mask=(iota == bb))
    pltpu.sync_copy(cntv.at[pl.ds(0, 8)], cnts_hbm.at[pl.ds(w * 8, 8)])

    @pl.when(w == 0)
    def _():
        for q in range(G // LANES):
            stg_s[pl.ds(q * LANES, LANES)] = zi
            stg_d[pl.ds(q * LANES, LANES)] = trash
        pltpu.sync_copy(stg_s.at[pl.ds(0, G)],
                        bsrc_hbm.at[pl.ds(DUMMY_OFF, G)])
        pltpu.sync_copy(stg_d.at[pl.ds(0, G)],
                        bdst_hbm.at[pl.ds(DUMMY_OFF, G)])


_bin_kernel = functools.partial(
    pl.kernel,
    out_type=(jax.ShapeDtypeStruct((TOTW,), jnp.int32),
              jax.ShapeDtypeStruct((TOTW,), jnp.int32),
              jax.ShapeDtypeStruct((NW * 8,), jnp.int32)),
    mesh=_mesh,
    compiler_params=pltpu.CompilerParams(needs_layout_passes=False),
    scratch_types=[
        pltpu.VMEM((2, G), jnp.int32),
        pltpu.VMEM((2, G), jnp.int32),
        pltpu.VMEM((K * STG,), jnp.int32),
        pltpu.VMEM((K * STG,), jnp.int32),
        pltpu.VMEM((LANES,), jnp.int32),
        pltpu.SemaphoreType.DMA,
    ],
)(_bin_body)


def _spmm_body(bsrc_hbm, bdst_hbm, cnts_hbm, u_hbm, agg_hbm, srcv, dstv,
               rows, zrows, cntv, accum, gsem, ssem):
    c = lax.axis_index("c")
    s = lax.axis_index("s")

    _zero_vmem_rows(zrows, 64)
    pltpu.sync_copy(cnts_hbm, cntv)

    def zero_accum():
        for t in range(NB // NS // 64):
            pltpu.sync_copy(zrows,
                            accum.at[pl.ds(s * (NB // NS) + t * 64, 64)])
        pltpu.sync_copy(zrows.at[pl.ds(0, 16)],
                        accum.at[pl.ds(s * (NB // NS) + (NB // NS // 64) * 64, 16)])

        @pl.when(s == 0)
        def _():
            pltpu.sync_copy(zrows.at[pl.ds(0, 16)], accum.at[pl.ds(NB, 16)])

    for kb in range(K // NC):
        bb = c * (K // NC) + kb
        lo = bb * NB
        zero_accum()
        plsc.subcore_barrier()

        t0 = s * 2
        n0 = cntv[pl.ds(t0 * 8 + bb, 1)][0]
        n1 = cntv[pl.ds((t0 + 1) * 8 + bb, 1)][0]
        ntot = n0 + n1
        npair = (ntot + 1) // 2

        def idx_off(jn):
            in0 = jn < n0
            tt = jnp.where(in0, t0, t0 + 1)
            jl = jnp.where(in0, jn, jn - n0)
            return jnp.where(jn < ntot, ((tt * K + bb) * CAPR + jl) * G,
                             DUMMY_OFF)

        for p in range(2):
            off = idx_off(p)
            pltpu.sync_copy(bsrc_hbm.at[pl.ds(off, G)], srcv.at[p])
            pltpu.sync_copy(bdst_hbm.at[pl.ds(off, G)], dstv.at[p])

        @pl.loop(0, npair)
        def _(jj):
            gd = [pltpu.async_copy(u_hbm.at[srcv.at[p]],
                                   rows.at[pl.ds(p * G, G)], gsem)
                  for p in range(2)]
            sd = []
            for p in range(2):
                gd[p].wait()
                sd.append(pltpu.async_copy(rows.at[pl.ds(p * G, G)],
                                           accum.at[dstv.at[p]], ssem,
                                           add=True))
            noff = [idx_off((jj + 1) * 2 + p) for p in range(2)]
            for p in range(2):
                pltpu.sync_copy(bsrc_hbm.at[pl.ds(noff[p], G)], srcv.at[p])
            for p in range(2):
                sd[p].wait()
                pltpu.sync_copy(bdst_hbm.at[pl.ds(noff[p], G)], dstv.at[p])

        plsc.subcore_barrier()
        for t in range(NB // NS // G):
            pltpu.sync_copy(accum.at[pl.ds(s * (NB // NS) + t * G, G)],
                            rows.at[pl.ds(0, G)])
            pltpu.sync_copy(rows.at[pl.ds(0, G)],
                            agg_hbm.at[pl.ds(lo + s * (NB // NS) + t * G, G)])
        tail = s * (NB // NS) + (NB // NS // G) * G
        pltpu.sync_copy(accum.at[pl.ds(tail, 16)], rows.at[pl.ds(0, 16)])
        pltpu.sync_copy(rows.at[pl.ds(0, 16)],
                        agg_hbm.at[pl.ds(lo + tail, 16)])
        if kb + 1 < K // NC:
            plsc.subcore_barrier()


_spmm_kernel = functools.partial(
    pl.kernel,
    out_type=jax.ShapeDtypeStruct((NPAD, H), jnp.float32),
    mesh=_mesh,
    scratch_types=[
        pltpu.VMEM((2, G), jnp.int32),
        pltpu.VMEM((2, G), jnp.int32),
        pltpu.VMEM((2 * G, H), jnp.float32),
        pltpu.VMEM((64, H), jnp.float32),
        pltpu.VMEM((NW * 8,), jnp.int32),
        pltpu.VMEM_SHARED((NB + 16, H), jnp.float32),
        pltpu.SemaphoreType.DMA,
        pltpu.SemaphoreType.DMA,
    ],
)(_spmm_body)


def _prep_body(x_ref, deg_ref, w_in_ref, b_in_ref, wc0_ref, u0_ref):
    dinv = lax.rsqrt(deg_ref[...] + 1.0)
    h0 = jnp.dot(x_ref[...], w_in_ref[...],
                 preferred_element_type=jnp.float32) + b_in_ref[...]
    u0_ref[...] = jnp.dot(h0, wc0_ref[...],
                          preferred_element_type=jnp.float32) * dinv


def _prep_call(x_pad, deg_col, W_in, b_in, Wc0):
    grid = (NPAD // RB,)
    return pl.pallas_call(
        _prep_body,
        grid=grid,
        in_specs=[
            pl.BlockSpec((RB, D_IN), lambda i: (i, 0)),
            pl.BlockSpec((RB, 1), lambda i: (i, 0)),
            pl.BlockSpec((D_IN, H), lambda i: (0, 0)),
            pl.BlockSpec((H,), lambda i: (0,)),
            pl.BlockSpec((H, H), lambda i: (0, 0)),
        ],
        out_specs=pl.BlockSpec((RB, H), lambda i: (i, 0)),
        out_shape=jax.ShapeDtypeStruct((NPAD, H), jnp.float32),
    )(x_pad, deg_col, W_in, b_in, Wc0)


def _post_body(has_res, has_next, agg_ref, u_ref, deg_ref, g_ref, bt_ref,
               bc_ref, *rest):
    idx = 0
    hprev_ref = rest[idx] if has_res else None
    idx += int(has_res)
    wcn_ref = rest[idx] if has_next else None
    idx += int(has_next)
    h_ref = rest[idx]
    un_ref = rest[idx + 1] if has_next else None

    dinv = lax.rsqrt(deg_ref[...] + 1.0)
    t = dinv * (agg_ref[...] + u_ref[...]) + bc_ref[...]
    mu = jnp.mean(t, axis=-1, keepdims=True)
    var = jnp.mean((t - mu) ** 2, axis=-1, keepdims=True)
    ln = (t - mu) * lax.rsqrt(var + 1e-5) * g_ref[...] + bt_ref[...]
    h = jnp.maximum(ln, 0.0)
    if has_res:
        h = h + hprev_ref[...]
    h_ref[...] = h
    if has_next:
        un_ref[...] = jnp.dot(h, wcn_ref[...],
                              preferred_element_type=jnp.float32) * dinv


def _post_call(agg, u, deg_col, gamma_i, beta_i, bc_i, h_prev=None,
               Wc_next=None):
    has_res = h_prev is not None
    has_next = Wc_next is not None
    grid = (NPAD // RB,)
    in_specs = [
        pl.BlockSpec((RB, H), lambda i: (i, 0)),
        pl.BlockSpec((RB, H), lambda i: (i, 0)),
        pl.BlockSpec((RB, 1), lambda i: (i, 0)),
        pl.BlockSpec((H,), lambda i: (0,)),
        pl.BlockSpec((H,), lambda i: (0,)),
        pl.BlockSpec((H,), lambda i: (0,)),
    ]
    args = [agg, u, deg_col, gamma_i, beta_i, bc_i]
    if has_res:
        in_specs.append(pl.BlockSpec((RB, H), lambda i: (i, 0)))
        args.append(h_prev)
    if has_next:
        in_specs.append(pl.BlockSpec((H, H), lambda i: (0, 0)))
        args.append(Wc_next)
    out_specs = [pl.BlockSpec((RB, H), lambda i: (i, 0))]
    out_shape = [jax.ShapeDtypeStruct((NPAD, H), jnp.float32)]
    if has_next:
        out_specs.append(pl.BlockSpec((RB, H), lambda i: (i, 0)))
        out_shape.append(jax.ShapeDtypeStruct((NPAD, H), jnp.float32))
    res = pl.pallas_call(
        functools.partial(_post_body, has_res, has_next),
        grid=grid,
        in_specs=in_specs,
        out_specs=out_specs,
        out_shape=out_shape,
    )(*args)
    return res if has_next else (res[0], None)


def kernel(x, edge_index, W_in, b_in, Wc, bc, gamma, beta):
    src = edge_index[0].astype(jnp.int32)
    dst = edge_index[1].astype(jnp.int32)
    src_p = jnp.concatenate(
        [src, jnp.zeros((EPAD - E,), jnp.int32)]).reshape(ROWS_ALL, G)
    dst_p = jnp.concatenate(
        [dst, jnp.full((EPAD - E,), NPAD, jnp.int32)]).reshape(ROWS_ALL, G)
    x_pad = jnp.pad(x, ((0, NPAD - N), (0, 0)))

    deg = _deg_kernel(dst_p)
    deg_col = deg.reshape(NPAD, 1)
    bsrc, bdst, cnts = _bin_kernel(src_p, dst_p)

    u = _prep_call(x_pad, deg_col, W_in, b_in, Wc[0])
    h = None
    for i in range(L):
        agg = _spmm_kernel(bsrc, bdst, cnts, u)
        h_prev = h if i > 0 else None
        Wc_next = Wc[i + 1] if i + 1 < L else None
        h, u = _post_call(agg, u, deg_col, gamma[i], beta[i], bc[i],
                          h_prev=h_prev, Wc_next=Wc_next)
    return h[:N]
